# Initial kernel scaffold; baseline (speedup 1.0000x reference)
import jax
import jax.numpy as jnp
from jax.experimental import pallas as pl


def kernel(x, edge_attr, edge_idx, eps0, We0, be0, W10, b10, W20, b20, eps1, We1, be1, W11, b11, W21, b21):
    raise NotImplementedError("write your pallas kernel here")


# speedup vs baseline: 2.5209x; 2.5209x over previous
import functools

import jax
import jax.numpy as jnp
from jax import lax
from jax.experimental import pallas as pl
from jax.experimental.pallas import tpu as pltpu
from jax.experimental.pallas import tpu_sc as plsc

N = 10000
E = 320000
D = 128
DE = 16

N_PAD = 10240
NW = 32
E_PER_W = E // NW
CHUNK = 80
N_CHUNKS = E_PER_W // CHUNK


def _edge_mlp_body(ea_ref, we_ref, be_ref, out_ref):
    out_ref[...] = (
        jnp.dot(ea_ref[...], we_ref[...], preferred_element_type=jnp.float32)
        + be_ref[...]
    )


def _edge_mlp(edge_attr, We, be):
    blk = 8000
    grid = E // blk
    return pl.pallas_call(
        _edge_mlp_body,
        grid=(grid,),
        in_specs=[
            pl.BlockSpec((blk, DE), lambda i: (i, 0)),
            pl.BlockSpec((DE, D), lambda i: (0, 0)),
            pl.BlockSpec((1, D), lambda i: (0, 0)),
        ],
        out_specs=pl.BlockSpec((blk, D), lambda i: (i, 0)),
        out_shape=jax.ShapeDtypeStruct((E, D), jnp.float32),
    )(edge_attr, We, be.reshape(1, D))


def _sc_msgpass_body(x_hbm, e_hbm, src_hbm, dst_hbm, out_hbm,
                     src_v, dst_v, xg_v, ev_v, zbuf, aggr, sem):
    c = lax.axis_index("c")
    s = lax.axis_index("s")
    wid = s * 2 + c
    base = wid * E_PER_W

    def zrow(i, _):
        for j in range(D // 16):
            zbuf[i, pl.ds(j * 16, 16)] = jnp.zeros((16,), jnp.float32)
        return 0
    lax.fori_loop(0, 32, zrow, 0)

    rows_per_sub = N_PAD // 16
    def zcp(t, _):
        pltpu.sync_copy(zbuf, aggr.at[pl.ds(s * rows_per_sub + t * 32, 32)])
        return 0
    lax.fori_loop(0, rows_per_sub // 32, zcp, 0)

    plsc.subcore_barrier()

    def chunk(g, _):
        off = base + g * CHUNK
        pltpu.sync_copy(src_hbm.at[pl.ds(off, CHUNK)], src_v)
        pltpu.sync_copy(dst_hbm.at[pl.ds(off, CHUNK)], dst_v)
        pltpu.async_copy(x_hbm.at[src_v], xg_v, sem).wait()
        pltpu.sync_copy(e_hbm.at[pl.ds(off, CHUNK)], ev_v)

        def row(i, _):
            for j in range(D // 16):
                sl = pl.ds(j * 16, 16)
                xg_v[i, sl] = jnp.maximum(xg_v[i, sl] + ev_v[i, sl], 0.0)
            return 0
        lax.fori_loop(0, CHUNK, row, 0)

        pltpu.sync_copy(xg_v, aggr.at[dst_v], add=True)
        return 0
    lax.fori_loop(0, N_CHUNKS, chunk, 0)

    plsc.subcore_barrier()

    def cpout(t, _):
        r = s * rows_per_sub + t * 64
        pltpu.sync_copy(aggr.at[pl.ds(r, 64)], out_hbm.at[c, pl.ds(r, 64)])
        return 0
    lax.fori_loop(0, rows_per_sub // 64, cpout, 0)


def _sc_msgpass(x, e, src, dst):
    mesh = plsc.VectorSubcoreMesh(core_axis_name="c", subcore_axis_name="s")
    f = pl.kernel(
        _sc_msgpass_body,
        out_type=jax.ShapeDtypeStruct((2, N_PAD, D), jnp.float32),
        mesh=mesh,
        scratch_types=[
            pltpu.VMEM((CHUNK,), jnp.int32),
            pltpu.VMEM((CHUNK,), jnp.int32),
            pltpu.VMEM((CHUNK, D), jnp.float32),
            pltpu.VMEM((CHUNK, D), jnp.float32),
            pltpu.VMEM((32, D), jnp.float32),
            pltpu.VMEM_SHARED((N_PAD, D), jnp.float32),
            pltpu.SemaphoreType.DMA,
        ],
    )
    return f(x, e, src, dst)


def _node_mlp_body(eps_ref, x_ref, p_ref, w1_ref, b1_ref, w2_ref, b2_ref,
                   out_ref):
    p = p_ref[0] + p_ref[1]
    h = (1.0 + eps_ref[0, 0]) * x_ref[...] + p
    t = jnp.dot(h, w1_ref[...], preferred_element_type=jnp.float32) + b1_ref[...]
    t = 0.5 * t * (1.0 + lax.erf(t * 0.7071067811865476))
    out_ref[...] = (
        jnp.dot(t, w2_ref[...], preferred_element_type=jnp.float32) + b2_ref[...]
    )


def _node_mlp(eps, x, partial, W1, b1, W2, b2):
    blk = 2000
    grid = N // blk
    return pl.pallas_call(
        _node_mlp_body,
        grid=(grid,),
        in_specs=[
            pl.BlockSpec(memory_space=pltpu.SMEM),
            pl.BlockSpec((blk, D), lambda i: (i, 0)),
            pl.BlockSpec((2, blk, D), lambda i: (0, i, 0)),
            pl.BlockSpec((D, D), lambda i: (0, 0)),
            pl.BlockSpec((1, D), lambda i: (0, 0)),
            pl.BlockSpec((D, D), lambda i: (0, 0)),
            pl.BlockSpec((1, D), lambda i: (0, 0)),
        ],
        out_specs=pl.BlockSpec((blk, D), lambda i: (i, 0)),
        out_shape=jax.ShapeDtypeStruct((N, D), jnp.float32),
    )(eps.reshape(1, 1), x, partial, W1, b1.reshape(1, D), W2, b2.reshape(1, D))


def kernel(x, edge_attr, edge_idx, eps0, We0, be0, W10, b10, W20, b20,
           eps1, We1, be1, W11, b11, W21, b21):
    src = edge_idx[0]
    dst = edge_idx[1]

    e0 = _edge_mlp(edge_attr, We0, be0)
    e1 = _edge_mlp(edge_attr, We1, be1)

    p0 = _sc_msgpass(x, e0, src, dst)
    h = _node_mlp(eps0, x, p0, W10, b10, W20, b20)

    p1 = _sc_msgpass(h, e1, src, dst)
    out = _node_mlp(eps1, h, p1, W11, b11, W21, b21)
    return out
